# baseline (speedup 1.0000x reference)
import functools

import jax
import jax.numpy as jnp
from jax import lax
from jax.experimental import pallas as pl
from jax.experimental.pallas import tpu as pltpu
from jax.experimental.pallas import tpu_sc as plsc

NUM_CORES = 2
NUM_SUBCORES = 16
NUM_WORKERS = NUM_CORES * NUM_SUBCORES
CB = 128
NS = 4
NT = 2


@functools.partial(jax.jit, static_argnames=("vocab", "dim"))
def _pack_table(tab_t, tail_pad, *, vocab, dim):
    nfull = vocab // 128
    iters = nfull // NUM_WORKERS + 1
    mesh = plsc.VectorSubcoreMesh(core_axis_name="c", subcore_axis_name="s")

    @functools.partial(
        pl.kernel,
        out_type=jax.ShapeDtypeStruct((vocab // 2, 128), jnp.float32),
        mesh=mesh,
        scratch_types=[
            pltpu.VMEM((2, dim, 128), jnp.float32),
            pltpu.VMEM((2, dim, 128), jnp.float32),
            [pltpu.SemaphoreType.DMA] * 2,
            [pltpu.SemaphoreType.DMA] * 2,
        ],
        compiler_params=pltpu.CompilerParams(
            use_tc_tiling_on_sc=True, needs_layout_passes=False),
    )
    def kt(tabt_hbm, tail_hbm, out_hbm, in_v, out_v, si, so):
        wid = lax.axis_index("s") * NUM_CORES + lax.axis_index("c")
        lanes = lax.iota(jnp.int32, 16)
        rots = [(lanes + r) & 15 for r in range(16)]

        def blk(mi):
            return wid + mi * NUM_WORKERS

        def i_copy(mi, s):
            m = blk(mi)
            return pltpu.make_async_copy(
                tabt_hbm.at[:, pl.ds(m * 128, 128)], in_v.at[s], si[s])

        def o_copy(mi, s):
            m = blk(mi)
            return pltpu.make_async_copy(
                out_v.at[s], out_hbm.at[pl.ds(m * 64, dim)], so[s])

        def transpose(s):
            def jbody(jh, carry):
                jg = jh // 2
                half = jh - jg * 2
                col2 = 2 * (jg * 16 + lanes) + half
                rowo = jg * 16 + lanes
                for cg in range(dim // 16):
                    for r in range(16):
                        chan = cg * 16 + rots[r]
                        vals = plsc.load_gather(in_v.at[s], [chan, col2])
                        plsc.store_scatter(
                            out_v.at[s], [rowo, half * 64 + chan], vals)
                return carry

            lax.fori_loop(0, 8, jbody, 0)

        def valid(mi):
            return blk(mi) < nfull

        def step(mi, s, first=False):
            if not first:
                @pl.when(valid(mi - 2))
                def _():
                    o_copy(mi - 2, s).wait()

            @pl.when(valid(mi))
            def _():
                i_copy(mi, s).wait()
                transpose(s)
                o_copy(mi, s).start()

            @pl.when(valid(mi + 2))
            def _():
                i_copy(mi + 2, s).start()

        @pl.when(valid(0))
        def _():
            i_copy(0, 0).start()

        @pl.when(valid(1))
        def _():
            i_copy(1, 1).start()

        step(0, 0, first=True)
        step(1, 1, first=True)

        def body(t, carry):
            step(2 * t, 0)
            step(2 * t + 1, 1)
            return carry

        lax.fori_loop(1, (iters - 1) // 2, body, 0)
        step(iters - 1, 0)

        @pl.when(valid(iters - 2))
        def _():
            o_copy(iters - 2, 1).wait()

        @pl.when(valid(iters - 1))
        def _():
            o_copy(iters - 1, 0).wait()

        @pl.when(wid == jnp.int32(0))
        def _():
            pltpu.sync_copy(tail_hbm, in_v.at[0])
            def tbody(j, carry):
                for half in range(2):
                    for c4 in range(dim // 16):
                        sl = pl.ds(c4 * 16, 16)
                        out_v[0, j, pl.ds(half * 64 + c4 * 16, 16)] = (
                            in_v[0, 2 * j + half, sl])
                return carry
            lax.fori_loop(0, 32, tbody, 0)
            pltpu.sync_copy(out_v.at[0, pl.ds(0, 32)],
                            out_hbm.at[pl.ds(nfull * 64, 32)])

    return kt(tab_t, tail_pad)


@functools.partial(jax.jit, static_argnames=("batch", "hist", "dim"))
def _embed_t(ids_t, table2, *, batch, hist, dim):
    upw = (batch // CB) // NUM_WORKERS
    steps = upw * hist
    assert steps % NS == 0 and steps // NS >= 4
    mesh = plsc.VectorSubcoreMesh(core_axis_name="c", subcore_axis_name="s")

    @functools.partial(
        pl.kernel,
        out_type=jax.ShapeDtypeStruct((hist, dim, batch), jnp.float32),
        mesh=mesh,
        scratch_types=[
            pltpu.VMEM((NS, CB), jnp.int32),
            pltpu.VMEM((NS, CB), jnp.int32),
            pltpu.VMEM((NS, CB), jnp.int32),
            pltpu.VMEM((NS, CB, 128), jnp.float32),
            pltpu.VMEM((NT, dim, CB), jnp.float32),
            [pltpu.SemaphoreType.DMA] * NS,
            [pltpu.SemaphoreType.DMA] * NS,
            [pltpu.SemaphoreType.DMA] * NT,
        ],
        compiler_params=pltpu.CompilerParams(
            use_tc_tiling_on_sc=True, needs_layout_passes=False),
    )
    def k(ids_hbm, tab_hbm, out_hbm, idx_v, idxg_v, par_v, rows_v, tr_v,
          si, sg, so):
        wid = lax.axis_index("s") * NUM_CORES + lax.axis_index("c")
        lanes = lax.iota(jnp.int32, 16)

        def hb(kk):
            u = kk // hist
            h = kk - u * hist
            b0 = (wid * upw + u) * CB
            return h, b0

        def i_copy(kk, s):
            h, b0 = hb(kk)
            return pltpu.make_async_copy(
                ids_hbm.at[h, pl.ds(b0, CB)], idx_v.at[s], si[s])

        def g_copy(s):
            return pltpu.make_async_copy(
                tab_hbm.at[idxg_v.at[s]], rows_v.at[s], sg[s])

        def s_copy(kk, t):
            h, b0 = hb(kk)
            return pltpu.make_async_copy(
                tr_v.at[t], out_hbm.at[h, :, pl.ds(b0, CB)], so[t])

        def split_ids(s):
            for bg in range(CB // 16):
                sl = pl.ds(bg * 16, 16)
                raw = idx_v[s, sl]
                idxg_v[s, sl] = lax.shift_right_logical(raw, jnp.int32(1))
                par_v[s, sl] = (raw & jnp.int32(1)) * jnp.int32(dim)

        rots = [(lanes + r) & 15 for r in range(16)]

        def compute(s, t):
            def mbody(m, carry2):
                bg = m // (dim // 16)
                cb = m - bg * (dim // 16)
                sl = pl.ds(bg * 16, 16)
                parc = par_v[s, sl] + cb * 16
                bcol = bg * 16 + lanes
                c0 = cb * 16
                for r in range(16):
                    vals = plsc.load_gather(rows_v.at[s],
                                            [bcol, parc + rots[r]])
                    plsc.store_scatter(tr_v.at[t], [c0 + rots[r], bcol],
                                       vals)
                return carry2

            lax.fori_loop(0, (CB // 16) * (dim // 16), mbody, 0)

        def step(kk, j, tj, first=False, refill_g=True, refill_i=True):
            g_copy(j).wait()
            if not first:
                s_copy(kk - NT, tj).wait()
            compute(j, tj)
            s_copy(kk, tj).start()
            if refill_g:
                i_copy(kk + NS, j).wait()
                split_ids(j)
                g_copy(j).start()
            if refill_i:
                i_copy(kk + 2 * NS, j).start()

        for j in range(NS):
            i_copy(j, j).start()
        for j in range(NS):
            i_copy(j, j).wait()
            split_ids(j)
            g_copy(j).start()
            i_copy(j + NS, j).start()

        for j in range(NS):
            step(j, j, j % NT, first=(j < NT))

        def body(q, carry):
            kk0 = q * NS
            for j in range(NS):
                step(kk0 + j, j, j % NT)
            return carry

        lax.fori_loop(1, steps // NS - 2, body, 0)

        for j in range(NS):
            kk = steps - 2 * NS + j
            step(kk, j, j % NT, refill_i=False)
        for j in range(NS):
            kk = steps - NS + j
            step(kk, j, j % NT, refill_g=False, refill_i=False)

        s_copy(steps - 2, 0).wait()
        s_copy(steps - 1, 1).wait()

    return k(ids_t, table2)


def kernel(token_ids, embedding_matrix):
    batch, hist = token_ids.shape
    vocab, dim = embedding_matrix.shape
    ids_t = token_ids.T.astype(jnp.int32)
    ntail = vocab % 128
    tail_pad = jnp.pad(embedding_matrix[vocab - ntail:],
                       ((0, 0), (0, 128 - dim)))
    table2 = _pack_table(embedding_matrix.T, tail_pad, vocab=vocab, dim=dim)
    out_t = _embed_t(ids_t, table2, batch=batch, hist=hist, dim=dim)
    return out_t.transpose(2, 0, 1)


# speedup vs baseline: 1.0963x; 1.0963x over previous
import functools

import jax
import jax.numpy as jnp
from jax import lax
from jax.experimental import pallas as pl
from jax.experimental.pallas import tpu as pltpu
from jax.experimental.pallas import tpu_sc as plsc

NUM_CORES = 2
NUM_SUBCORES = 16
NUM_WORKERS = NUM_CORES * NUM_SUBCORES
CB = 128
NS = 4
NT = 2


@functools.partial(jax.jit, static_argnames=("vocab", "dim"))
def _pack_table(tab_t, tail_pad, *, vocab, dim):
    nfull = vocab // 128
    per_w = (nfull // (2 * NUM_WORKERS)) * 2
    nslab = per_w // 2
    rest = nfull - per_w * NUM_WORKERS
    mesh = plsc.VectorSubcoreMesh(core_axis_name="c", subcore_axis_name="s")

    @functools.partial(
        pl.kernel,
        out_type=jax.ShapeDtypeStruct((vocab // 2, 128), jnp.float32),
        mesh=mesh,
        scratch_types=[
            pltpu.VMEM((2, dim, 256), jnp.float32),
            pltpu.VMEM((2, 2 * dim, 128), jnp.float32),
            [pltpu.SemaphoreType.DMA] * 2,
            [pltpu.SemaphoreType.DMA] * 2,
        ],
        compiler_params=pltpu.CompilerParams(
            use_tc_tiling_on_sc=True, needs_layout_passes=False),
    )
    def kt(tabt_hbm, tail_hbm, out_hbm, in_v, out_v, si, so):
        wid = lax.axis_index("s") * NUM_CORES + lax.axis_index("c")
        lanes = lax.iota(jnp.int32, 16)
        rots = [(lanes + r) & 15 for r in range(16)]

        def i_copy(m0, s):
            return pltpu.make_async_copy(
                tabt_hbm.at[:, pl.ds(m0 * 128, 256)], in_v.at[s], si[s])

        def o_copy(m0, s):
            return pltpu.make_async_copy(
                out_v.at[s], out_hbm.at[pl.ds(m0 * 64, 2 * dim)], so[s])

        def transpose(s):
            def jbody(jh, carry):
                jg = jh // 2
                half = jh - jg * 2
                col2 = 2 * (jg * 16 + lanes) + half
                rowo = jg * 16 + lanes
                for cg in range(dim // 16):
                    for r in range(16):
                        chan = cg * 16 + rots[r]
                        vals = plsc.load_gather(in_v.at[s], [chan, col2])
                        plsc.store_scatter(
                            out_v.at[s], [rowo, half * 64 + chan], vals)
                return carry

            lax.fori_loop(0, 16, jbody, 0)

        base = wid * per_w

        def m0_of(t):
            return base + 2 * t

        def step(t, s, first=False):
            if not first:
                o_copy(m0_of(t - 2), s).wait()
            i_copy(m0_of(t), s).wait()
            transpose(s)
            o_copy(m0_of(t), s).start()

        i_copy(m0_of(0), 0).start()
        i_copy(m0_of(1), 1).start()

        step(0, 0, first=True)
        i_copy(m0_of(2), 0).start()
        step(1, 1, first=True)
        i_copy(m0_of(3), 1).start()

        def body(t, carry):
            step(2 * t, 0)
            i_copy(m0_of(2 * t + 2), 0).start()
            step(2 * t + 1, 1)
            i_copy(m0_of(2 * t + 3), 1).start()
            return carry

        lax.fori_loop(1, nslab // 2 - 1, body, 0)
        step(nslab - 2, 0)
        step(nslab - 1, 1)
        o_copy(m0_of(nslab - 2), 0).wait()
        o_copy(m0_of(nslab - 1), 1).wait()

        @pl.when(wid < jnp.int32(rest // 2))
        def _():
            m0 = per_w * NUM_WORKERS + wid * 2
            i_copy(m0, 0).start()
            i_copy(m0, 0).wait()
            transpose(0)
            o_copy(m0, 0).start()
            o_copy(m0, 0).wait()

        @pl.when(wid == jnp.int32(0))
        def _():
            pltpu.sync_copy(tail_hbm, in_v.at[0, :, pl.ds(0, 128)])

            def tbody(j, carry):
                for half in range(2):
                    for c4 in range(dim // 16):
                        sl = pl.ds(c4 * 16, 16)
                        out_v[0, j, pl.ds(half * 64 + c4 * 16, 16)] = (
                            in_v[0, 2 * j + half, sl])
                return carry

            lax.fori_loop(0, 32, tbody, 0)
            pltpu.sync_copy(out_v.at[0, pl.ds(0, 32)],
                            out_hbm.at[pl.ds(nfull * 64, 32)])

    return kt(tab_t, tail_pad)


@functools.partial(jax.jit, static_argnames=("batch", "hist", "dim"))
def _embed_t(ids_t, table2, *, batch, hist, dim):
    upw = (batch // CB) // NUM_WORKERS
    steps = upw * hist
    assert steps % NS == 0 and steps // NS >= 4
    mesh = plsc.VectorSubcoreMesh(core_axis_name="c", subcore_axis_name="s")

    @functools.partial(
        pl.kernel,
        out_type=jax.ShapeDtypeStruct((hist, dim, batch), jnp.float32),
        mesh=mesh,
        scratch_types=[
            pltpu.VMEM((NS, CB), jnp.int32),
            pltpu.VMEM((NS, CB), jnp.int32),
            pltpu.VMEM((NS, CB), jnp.int32),
            pltpu.VMEM((NS, CB, 128), jnp.float32),
            pltpu.VMEM((NT, dim, CB), jnp.float32),
            [pltpu.SemaphoreType.DMA] * NS,
            [pltpu.SemaphoreType.DMA] * NS,
            [pltpu.SemaphoreType.DMA] * NT,
        ],
        compiler_params=pltpu.CompilerParams(
            use_tc_tiling_on_sc=True, needs_layout_passes=False),
    )
    def k(ids_hbm, tab_hbm, out_hbm, idx_v, idxg_v, par_v, rows_v, tr_v,
          si, sg, so):
        wid = lax.axis_index("s") * NUM_CORES + lax.axis_index("c")
        lanes = lax.iota(jnp.int32, 16)

        def hb(kk):
            u = kk // hist
            h = kk - u * hist
            b0 = (wid * upw + u) * CB
            return h, b0

        def i_copy(kk, s):
            h, b0 = hb(kk)
            return pltpu.make_async_copy(
                ids_hbm.at[h, pl.ds(b0, CB)], idx_v.at[s], si[s])

        def g_copy(s):
            return pltpu.make_async_copy(
                tab_hbm.at[idxg_v.at[s]], rows_v.at[s], sg[s])

        def s_copy(kk, t):
            h, b0 = hb(kk)
            return pltpu.make_async_copy(
                tr_v.at[t], out_hbm.at[h, :, pl.ds(b0, CB)], so[t])

        def split_ids(s):
            for bg in range(CB // 16):
                sl = pl.ds(bg * 16, 16)
                raw = idx_v[s, sl]
                idxg_v[s, sl] = lax.shift_right_logical(raw, jnp.int32(1))
                par_v[s, sl] = (raw & jnp.int32(1)) * jnp.int32(dim)

        rots = [(lanes + r) & 15 for r in range(16)]

        def compute(s, t):
            def mbody(m, carry2):
                bg = m // (dim // 16)
                cb = m - bg * (dim // 16)
                sl = pl.ds(bg * 16, 16)
                parc = par_v[s, sl] + cb * 16
                bcol = bg * 16 + lanes
                c0 = cb * 16
                for r in range(16):
                    vals = plsc.load_gather(rows_v.at[s],
                                            [bcol, parc + rots[r]])
                    plsc.store_scatter(tr_v.at[t], [c0 + rots[r], bcol],
                                       vals)
                return carry2

            lax.fori_loop(0, (CB // 16) * (dim // 16), mbody, 0)

        def step(kk, j, tj, first=False, refill_g=True, refill_i=True):
            g_copy(j).wait()
            if not first:
                s_copy(kk - NT, tj).wait()
            compute(j, tj)
            s_copy(kk, tj).start()
            if refill_g:
                i_copy(kk + NS, j).wait()
                split_ids(j)
                g_copy(j).start()
            if refill_i:
                i_copy(kk + 2 * NS, j).start()

        for j in range(NS):
            i_copy(j, j).start()
        for j in range(NS):
            i_copy(j, j).wait()
            split_ids(j)
            g_copy(j).start()
            i_copy(j + NS, j).start()

        for j in range(NS):
            step(j, j, j % NT, first=(j < NT))

        def body(q, carry):
            kk0 = q * NS
            for j in range(NS):
                step(kk0 + j, j, j % NT)
            return carry

        lax.fori_loop(1, steps // NS - 2, body, 0)

        for j in range(NS):
            kk = steps - 2 * NS + j
            step(kk, j, j % NT, refill_i=False)
        for j in range(NS):
            kk = steps - NS + j
            step(kk, j, j % NT, refill_g=False, refill_i=False)

        s_copy(steps - 2, 0).wait()
        s_copy(steps - 1, 1).wait()

    return k(ids_t, table2)


def kernel(token_ids, embedding_matrix):
    batch, hist = token_ids.shape
    vocab, dim = embedding_matrix.shape
    ids_t = token_ids.T.astype(jnp.int32)
    ntail = vocab % 128
    tail_pad = jnp.pad(embedding_matrix[vocab - ntail:],
                       ((0, 0), (0, 128 - dim)))
    table2 = _pack_table(embedding_matrix.T, tail_pad, vocab=vocab, dim=dim)
    out_t = _embed_t(ids_t, table2, batch=batch, hist=hist, dim=dim)
    return out_t.transpose(2, 0, 1)


# speedup vs baseline: 1.2057x; 1.0998x over previous
import functools

import jax
import jax.numpy as jnp
from jax import lax
from jax.experimental import pallas as pl
from jax.experimental.pallas import tpu as pltpu
from jax.experimental.pallas import tpu_sc as plsc

NUM_CORES = 2
NUM_SUBCORES = 16
NUM_WORKERS = NUM_CORES * NUM_SUBCORES
CB = 128
NS = 4
NT = 2


@functools.partial(jax.jit, static_argnames=("batch", "hist", "dim"))
def _embed_t(ids_t, table2, *, batch, hist, dim):
    upw = (batch // CB) // NUM_WORKERS
    steps = upw * hist
    assert steps % NS == 0 and steps // NS >= 4
    mesh = plsc.VectorSubcoreMesh(core_axis_name="c", subcore_axis_name="s")

    @functools.partial(
        pl.kernel,
        out_type=jax.ShapeDtypeStruct((hist, dim, batch), jnp.float32),
        mesh=mesh,
        scratch_types=[
            pltpu.VMEM((NS, CB), jnp.int32),
            pltpu.VMEM((NS, CB), jnp.int32),
            pltpu.VMEM((NS, CB), jnp.int32),
            pltpu.VMEM((NS, CB, 128), jnp.float32),
            pltpu.VMEM((NT, dim, CB), jnp.float32),
            [pltpu.SemaphoreType.DMA] * NS,
            [pltpu.SemaphoreType.DMA] * NS,
            [pltpu.SemaphoreType.DMA] * NT,
        ],
        compiler_params=pltpu.CompilerParams(
            use_tc_tiling_on_sc=True, needs_layout_passes=False),
    )
    def k(ids_hbm, tab_hbm, out_hbm, idx_v, idxg_v, par_v, rows_v, tr_v,
          si, sg, so):
        wid = lax.axis_index("s") * NUM_CORES + lax.axis_index("c")
        lanes = lax.iota(jnp.int32, 16)

        def hb(kk):
            u = kk // hist
            h = kk - u * hist
            b0 = (wid * upw + u) * CB
            return h, b0

        def i_copy(kk, s):
            h, b0 = hb(kk)
            return pltpu.make_async_copy(
                ids_hbm.at[h, pl.ds(b0, CB)], idx_v.at[s], si[s])

        def g_copy(s):
            return pltpu.make_async_copy(
                tab_hbm.at[idxg_v.at[s]], rows_v.at[s], sg[s])

        def s_copy(kk, t):
            h, b0 = hb(kk)
            return pltpu.make_async_copy(
                tr_v.at[t], out_hbm.at[h, :, pl.ds(b0, CB)], so[t])

        def split_ids(s):
            for bg in range(CB // 16):
                sl = pl.ds(bg * 16, 16)
                raw = idx_v[s, sl]
                idxg_v[s, sl] = lax.shift_right_logical(raw, jnp.int32(1))
                par_v[s, sl] = (raw & jnp.int32(1)) * jnp.int32(dim)

        rots = [(lanes + r) & 15 for r in range(16)]

        def compute(s, t):
            def mbody(m, carry2):
                bg = m // (dim // 16)
                cb = m - bg * (dim // 16)
                sl = pl.ds(bg * 16, 16)
                parc = par_v[s, sl] + cb * 16
                bcol = bg * 16 + lanes
                c0 = cb * 16
                for r in range(16):
                    vals = plsc.load_gather(rows_v.at[s],
                                            [bcol, parc + rots[r]])
                    plsc.store_scatter(tr_v.at[t], [c0 + rots[r], bcol],
                                       vals)
                return carry2

            lax.fori_loop(0, (CB // 16) * (dim // 16), mbody, 0)

        def step(kk, j, tj, first=False, refill_g=True, refill_i=True):
            g_copy(j).wait()
            if not first:
                s_copy(kk - NT, tj).wait()
            compute(j, tj)
            s_copy(kk, tj).start()
            if refill_g:
                i_copy(kk + NS, j).wait()
                split_ids(j)
                g_copy(j).start()
            if refill_i:
                i_copy(kk + 2 * NS, j).start()

        for j in range(NS):
            i_copy(j, j).start()
        for j in range(NS):
            i_copy(j, j).wait()
            split_ids(j)
            g_copy(j).start()
            i_copy(j + NS, j).start()

        for j in range(NS):
            step(j, j, j % NT, first=(j < NT))

        def body(q, carry):
            kk0 = q * NS
            for j in range(NS):
                step(kk0 + j, j, j % NT)
            return carry

        lax.fori_loop(1, steps // NS - 2, body, 0)

        for j in range(NS):
            kk = steps - 2 * NS + j
            step(kk, j, j % NT, refill_i=False)
        for j in range(NS):
            kk = steps - NS + j
            step(kk, j, j % NT, refill_g=False, refill_i=False)

        s_copy(steps - 2, 0).wait()
        s_copy(steps - 1, 1).wait()

    return k(ids_t, table2)


def kernel(token_ids, embedding_matrix):
    batch, hist = token_ids.shape
    dim = embedding_matrix.shape[1]
    ids_t = token_ids.T.astype(jnp.int32)
    table2 = embedding_matrix.reshape(-1, 128)
    out_t = _embed_t(ids_t, table2, batch=batch, hist=hist, dim=dim)
    return out_t.transpose(2, 0, 1)


# speedup vs baseline: 1.4824x; 1.2295x over previous
import functools

import jax
import jax.numpy as jnp
from jax import lax
from jax.experimental import pallas as pl
from jax.experimental.pallas import tpu as pltpu
from jax.experimental.pallas import tpu_sc as plsc

NUM_CORES = 2
NUM_SUBCORES = 16
NUM_WORKERS = NUM_CORES * NUM_SUBCORES
CB = 128
NS = 4
NT = 2


@functools.partial(jax.jit, static_argnames=("batch", "hist", "dim"))
def _embed_t(ids_t, table2, *, batch, hist, dim):
    upw = (batch // CB) // NUM_WORKERS
    steps = upw * hist
    assert steps % NS == 0 and steps // NS >= 4
    mesh = plsc.VectorSubcoreMesh(core_axis_name="c", subcore_axis_name="s")

    @functools.partial(
        pl.kernel,
        out_type=jax.ShapeDtypeStruct((hist, dim, batch), jnp.float32),
        mesh=mesh,
        scratch_types=[
            pltpu.VMEM((NS, CB), jnp.int32),
            pltpu.VMEM((NS, CB), jnp.int32),
            pltpu.VMEM((NS, CB), jnp.int32),
            pltpu.VMEM((NS, CB, 128), jnp.float32),
            pltpu.VMEM((NT, dim, CB), jnp.float32),
            [pltpu.SemaphoreType.DMA] * NS,
            [pltpu.SemaphoreType.DMA] * NS,
            [pltpu.SemaphoreType.DMA] * NT,
        ],
        compiler_params=pltpu.CompilerParams(
            use_tc_tiling_on_sc=True, needs_layout_passes=False),
    )
    def k(ids_hbm, tab_hbm, out_hbm, idx_v, idxg_v, par_v, rows_v, tr_v,
          si, sg, so):
        wid = lax.axis_index("s") * NUM_CORES + lax.axis_index("c")
        lanes = lax.iota(jnp.int32, 16)

        def hb(kk):
            u = kk // hist
            h = kk - u * hist
            b0 = (wid * upw + u) * CB
            return h, b0

        def i_copy(kk, s):
            h, b0 = hb(kk)
            return pltpu.make_async_copy(
                ids_hbm.at[h, pl.ds(b0, CB)], idx_v.at[s], si[s])

        def g_copy(s):
            return pltpu.make_async_copy(
                tab_hbm.at[idxg_v.at[s]], rows_v.at[s], sg[s])

        def s_copy(kk, t):
            h, b0 = hb(kk)
            return pltpu.make_async_copy(
                tr_v.at[t], out_hbm.at[h, :, pl.ds(b0, CB)], so[t])

        def split_ids(s):
            for bg in range(CB // 16):
                sl = pl.ds(bg * 16, 16)
                raw = idx_v[s, sl]
                idxg_v[s, sl] = lax.shift_right_logical(raw, jnp.int32(1))
                par_v[s, sl] = (raw & jnp.int32(1)) * jnp.int32(dim)

        rots = [(lanes + r) & 15 for r in range(16)]

        def compute(s, t):
            @plsc.parallel_loop(0, (CB // 16) * (dim // 16), unroll=2)
            def _(m):
                bg = m // (dim // 16)
                cb = m - bg * (dim // 16)
                sl = pl.ds(bg * 16, 16)
                parc = par_v[s, sl] + cb * 16
                bcol = bg * 16 + lanes
                c0 = cb * 16
                for r in range(16):
                    vals = plsc.load_gather(rows_v.at[s],
                                            [bcol, parc + rots[r]])
                    plsc.store_scatter(tr_v.at[t], [c0 + rots[r], bcol],
                                       vals)

        def step(kk, j, tj, first=False, refill_g=True, refill_i=True):
            g_copy(j).wait()
            if not first:
                s_copy(kk - NT, tj).wait()
            compute(j, tj)
            s_copy(kk, tj).start()
            if refill_g:
                i_copy(kk + NS, j).wait()
                split_ids(j)
                g_copy(j).start()
            if refill_i:
                i_copy(kk + 2 * NS, j).start()

        for j in range(NS):
            i_copy(j, j).start()
        for j in range(NS):
            i_copy(j, j).wait()
            split_ids(j)
            g_copy(j).start()
            i_copy(j + NS, j).start()

        for j in range(NS):
            step(j, j, j % NT, first=(j < NT))

        def body(q, carry):
            kk0 = q * NS
            for j in range(NS):
                step(kk0 + j, j, j % NT)
            return carry

        lax.fori_loop(1, steps // NS - 2, body, 0)

        for j in range(NS):
            kk = steps - 2 * NS + j
            step(kk, j, j % NT, refill_i=False)
        for j in range(NS):
            kk = steps - NS + j
            step(kk, j, j % NT, refill_g=False, refill_i=False)

        s_copy(steps - 2, 0).wait()
        s_copy(steps - 1, 1).wait()

    return k(ids_t, table2)


def kernel(token_ids, embedding_matrix):
    batch, hist = token_ids.shape
    dim = embedding_matrix.shape[1]
    ids_t = token_ids.T.astype(jnp.int32)
    table2 = embedding_matrix.reshape(-1, 128)
    out_t = _embed_t(ids_t, table2, batch=batch, hist=hist, dim=dim)
    return out_t.transpose(2, 0, 1)
